# Initial kernel scaffold; baseline (speedup 1.0000x reference)
import jax
import jax.numpy as jnp
from jax.experimental import pallas as pl


def kernel(x, segment_ids, W_score, b_score, W_read, b_read):
    raise NotImplementedError("write your pallas kernel here")


# speedup vs baseline: 10.0505x; 10.0505x over previous
import functools

import jax
import jax.numpy as jnp
from jax import lax
from jax.experimental import pallas as pl
from jax.experimental.pallas import tpu as pltpu
from jax.experimental.pallas import tpu_sc as plsc

N = 320000
D = 128
NSEG = 10000
NWORK = 32
SPT = 320
NSEG_PAD = NWORK * SPT
CH = 256
NV = D // 16


def _sc_segment_softmax_sum(x_flat, ids32, row_bounds, w_flat):
    mesh = plsc.VectorSubcoreMesh(core_axis_name="c", subcore_axis_name="s")

    @functools.partial(
        pl.kernel,
        mesh=mesh,
        out_type=jax.ShapeDtypeStruct((NSEG_PAD * D,), jnp.float32),
        scratch_types=[
            pltpu.VMEM((CH * D,), jnp.float32),
            pltpu.VMEM((CH + 16,), jnp.int32),
            pltpu.VMEM((48,), jnp.int32),
            pltpu.VMEM((D,), jnp.float32),
            pltpu.VMEM((SPT * D,), jnp.float32),
        ],
    )
    def k(x_hbm, ids_hbm, rb_hbm, w_hbm, out_hbm, xbuf, idbuf, rb, wbuf, outb):
        wid = lax.axis_index("s") * 2 + lax.axis_index("c")
        pltpu.sync_copy(rb_hbm, rb)
        pltpu.sync_copy(w_hbm, wbuf)

        seg_lo = wid * SPT
        rbv = rb[pl.ds(wid, 16)]
        row_lo = rbv[0]
        row_hi = rbv[1]

        zero16 = jnp.zeros((16,), jnp.float32)

        def zrow(i, _):
            outb[pl.ds(i * 16, 16)] = zero16
            return 0

        lax.fori_loop(0, SPT * NV, zrow, 0)

        ws = [wbuf[pl.ds(kk * 16, 16)] for kk in range(NV)]
        lane = lax.iota(jnp.int32, 16)
        perms = [lane ^ st for st in (8, 4, 2, 1)]

        aligned_lo = (row_lo // 8) * 8
        nchunks = (row_hi - aligned_lo + CH - 1) // CH

        def flush(vs, dsum, cs):
            inv = 1.0 / dsum
            base = (cs - seg_lo) * D
            for kk in range(NV):
                outb[pl.ds(base + kk * 16, 16)] = vs[kk] * inv

        def chunk(g, carry):
            start = aligned_lo + g * CH
            s_g = jnp.minimum(start, N - CH)
            pltpu.sync_copy(x_hbm.at[pl.ds(s_g * D, CH * D)], xbuf)
            pltpu.sync_copy(ids_hbm.at[pl.ds(s_g, CH)], idbuf.at[pl.ds(0, CH)])
            lo_g = jnp.maximum(row_lo, start)
            hi_g = jnp.minimum(start + CH, row_hi)

            def row(r, c2):
                m = c2[0]
                dsum = c2[1]
                vs = c2[2:2 + NV]
                cs = c2[2 + NV]
                off = r - s_g
                xbase = off * D
                xs = [xbuf[pl.ds(xbase + kk * 16, 16)] for kk in range(NV)]
                p = [xs[kk] * ws[kk] for kk in range(NV)]
                t4 = [p[2 * kk] + p[2 * kk + 1] for kk in range(NV // 2)]
                t2 = [t4[0] + t4[1], t4[2] + t4[3]]
                tt = t2[0] + t2[1]
                for pm in perms:
                    tt = tt + tt.at[pm].get(mode="promise_in_bounds")
                sv = tt
                sid = idbuf[pl.ds(off, 16)][0]
                changed = sid != cs

                @pl.when(changed & (cs >= 0))
                def _():
                    flush(vs, dsum, cs)

                m2 = jnp.where(changed, sv, jnp.maximum(m, sv))
                cf = jnp.where(changed, jnp.float32(0.0), jnp.exp(m - m2))
                wgt = jnp.exp(sv - m2)
                d2 = dsum * cf + wgt
                v2 = tuple(vs[kk] * cf + wgt * xs[kk] for kk in range(NV))
                return (m2, d2) + v2 + (sid,)

            return lax.fori_loop(lo_g, hi_g, row, carry)

        init = (
            (jnp.full((16,), -3.0e38, jnp.float32),
             jnp.full((16,), 1.0, jnp.float32))
            + tuple(zero16 for _ in range(NV))
            + (jnp.int32(-1),)
        )
        fin = lax.fori_loop(0, nchunks, chunk, init)

        @pl.when(fin[2 + NV] >= 0)
        def _():
            flush(fin[2:2 + NV], fin[1], fin[2 + NV])

        pltpu.sync_copy(outb, out_hbm.at[pl.ds(seg_lo * D, SPT * D)])

    return k(x_flat, ids32, row_bounds, w_flat)


def _tc_readout(sx, W_read, b_row):
    def mm(sx_ref, w_ref, b_ref, o_ref):
        o_ref[...] = (
            jnp.dot(sx_ref[...], w_ref[...], preferred_element_type=jnp.float32)
            + b_ref[...]
        )

    return pl.pallas_call(
        mm,
        out_shape=jax.ShapeDtypeStruct((NSEG, D), jnp.float32),
        grid=(25,),
        in_specs=[
            pl.BlockSpec((400, D), lambda i: (i, 0)),
            pl.BlockSpec((D, D), lambda i: (0, 0)),
            pl.BlockSpec((1, D), lambda i: (0, 0)),
        ],
        out_specs=pl.BlockSpec((400, D), lambda i: (i, 0)),
    )(sx, W_read, b_row)


@jax.jit
def kernel(x, segment_ids, W_score, b_score, W_read, b_read):
    del b_score
    ids32 = segment_ids.astype(jnp.int32)
    seg_bounds = jnp.minimum(jnp.arange(33, dtype=jnp.int32) * SPT, NSEG)
    rb = jnp.searchsorted(ids32, seg_bounds, side="left").astype(jnp.int32)
    rb = jnp.concatenate([rb, jnp.full((15,), N, jnp.int32)])
    sx_flat = _sc_segment_softmax_sum(
        x.reshape(N * D), ids32, rb, W_score.reshape(D)
    )
    sx = sx_flat.reshape(NSEG_PAD, D)[:NSEG]
    return _tc_readout(sx, W_read, b_read.reshape(1, D))


# speedup vs baseline: 11.5239x; 1.1466x over previous
import functools

import jax
import jax.numpy as jnp
from jax import lax
from jax.experimental import pallas as pl
from jax.experimental.pallas import tpu as pltpu
from jax.experimental.pallas import tpu_sc as plsc

N = 320000
D = 128
NSEG = 10000
NWORK = 32
SPT = 320
NSEG_PAD = NWORK * SPT
CH = 256
NV = D // 16


def _sc_segment_softmax_sum(x_flat, ids32, row_bounds, w_flat):
    mesh = plsc.VectorSubcoreMesh(core_axis_name="c", subcore_axis_name="s")

    @functools.partial(
        pl.kernel,
        mesh=mesh,
        out_type=jax.ShapeDtypeStruct((NSEG_PAD * D,), jnp.float32),
        scratch_types=[
            pltpu.VMEM((CH * D,), jnp.float32),
            pltpu.VMEM((CH * D,), jnp.float32),
            pltpu.VMEM((CH + 16,), jnp.int32),
            pltpu.VMEM((CH + 16,), jnp.int32),
            pltpu.VMEM((48,), jnp.int32),
            pltpu.VMEM((D,), jnp.float32),
            pltpu.VMEM((SPT * D,), jnp.float32),
            pltpu.SemaphoreType.DMA,
            pltpu.SemaphoreType.DMA,
        ],
    )
    def k(x_hbm, ids_hbm, rb_hbm, w_hbm, out_hbm,
          xbuf0, xbuf1, idb0, idb1, rb, wbuf, outb, sem0, sem1):
        wid = lax.axis_index("s") * 2 + lax.axis_index("c")
        pltpu.sync_copy(rb_hbm, rb)
        pltpu.sync_copy(w_hbm, wbuf)

        seg_lo = wid * SPT
        rbv = rb[pl.ds(wid, 16)]
        row_lo = rbv[0]
        row_hi = rbv[1]

        zero16 = jnp.zeros((16,), jnp.float32)

        def zrow(i, _):
            outb[pl.ds(i * 16, 16)] = zero16
            return 0

        lax.fori_loop(0, SPT * NV, zrow, 0)

        ws = [wbuf[pl.ds(kk * 16, 16)] for kk in range(NV)]
        lane = lax.iota(jnp.int32, 16)
        perms = [lane ^ st for st in (8, 4, 2, 1)]

        aligned_lo = (row_lo // 8) * 8
        nchunks = (row_hi - aligned_lo + CH - 1) // CH

        def flush(vs, dsum, cs):
            inv = 1.0 / dsum
            base = (cs - seg_lo) * D
            for kk in range(NV):
                outb[pl.ds(base + kk * 16, 16)] = vs[kk] * inv

        xbufs = [xbuf0, xbuf1]
        idbs = [idb0, idb1]
        sems = [sem0, sem1]

        def issue(g, b):
            s_g = jnp.minimum(aligned_lo + g * CH, N - CH)
            pltpu.async_copy(x_hbm.at[pl.ds(s_g * D, CH * D)], xbufs[b], sems[b])
            pltpu.async_copy(
                ids_hbm.at[pl.ds(s_g, CH)], idbs[b].at[pl.ds(0, CH)], sems[b]
            )

        def wait(b):
            pltpu.make_async_copy(
                x_hbm.at[pl.ds(0, CH * D)], xbufs[b], sems[b]
            ).wait()
            pltpu.make_async_copy(
                ids_hbm.at[pl.ds(0, CH)], idbs[b].at[pl.ds(0, CH)], sems[b]
            ).wait()

        def process(g, b, carry):
            start = aligned_lo + g * CH
            s_g = jnp.minimum(start, N - CH)
            lo_g = jnp.maximum(row_lo, start)
            hi_g = jnp.minimum(start + CH, row_hi)
            xb = xbufs[b]
            ib = idbs[b]

            def row(r, c2):
                m = c2[0]
                dsum = c2[1]
                vs = c2[2:2 + NV]
                cs = c2[2 + NV]
                off = r - s_g
                xbase = off * D
                xs = [xb[pl.ds(xbase + kk * 16, 16)] for kk in range(NV)]
                p = [xs[kk] * ws[kk] for kk in range(NV)]
                t4 = [p[2 * kk] + p[2 * kk + 1] for kk in range(NV // 2)]
                t2 = [t4[0] + t4[1], t4[2] + t4[3]]
                tt = t2[0] + t2[1]
                for pm in perms:
                    tt = tt + tt.at[pm].get(mode="promise_in_bounds")
                sv = tt
                sid = ib[pl.ds(off, 16)][0]
                changed = sid != cs

                @pl.when(changed & (cs >= 0))
                def _():
                    flush(vs, dsum, cs)

                m2 = jnp.where(changed, sv, jnp.maximum(m, sv))
                cf = jnp.where(changed, jnp.float32(0.0), jnp.exp(m - m2))
                wgt = jnp.exp(sv - m2)
                d2 = dsum * cf + wgt
                v2 = tuple(vs[kk] * cf + wgt * xs[kk] for kk in range(NV))
                return (m2, d2) + v2 + (sid,)

            n4 = jnp.maximum(hi_g - lo_g, 0) // 4

            def quad(q, c2):
                r0 = lo_g + q * 4
                for j in range(4):
                    c2 = row(r0 + j, c2)
                return c2

            c = lax.fori_loop(0, n4, quad, carry)
            return lax.fori_loop(lo_g + n4 * 4, hi_g, row, c)

        init = (
            (jnp.full((16,), -3.0e38, jnp.float32),
             jnp.full((16,), 1.0, jnp.float32))
            + tuple(zero16 for _ in range(NV))
            + (jnp.int32(-1),)
        )

        issue(0, 0)
        npairs = (nchunks + 1) // 2

        def pair(gp, carry):
            g0 = gp * 2
            issue(g0 + 1, 1)
            wait(0)
            carry = process(g0, 0, carry)
            issue(g0 + 2, 0)
            wait(1)
            carry = process(g0 + 1, 1, carry)
            return carry

        fin = lax.fori_loop(0, npairs, pair, init)
        wait(0)

        @pl.when(fin[2 + NV] >= 0)
        def _():
            flush(fin[2:2 + NV], fin[1], fin[2 + NV])

        pltpu.sync_copy(outb, out_hbm.at[pl.ds(seg_lo * D, SPT * D)])

    return k(x_flat, ids32, row_bounds, w_flat)


def _tc_readout(sx, W_read, b_row):
    def mm(sx_ref, w_ref, b_ref, o_ref):
        o_ref[...] = (
            jnp.dot(sx_ref[...], w_ref[...], preferred_element_type=jnp.float32)
            + b_ref[...]
        )

    return pl.pallas_call(
        mm,
        out_shape=jax.ShapeDtypeStruct((NSEG, D), jnp.float32),
        grid=(25,),
        in_specs=[
            pl.BlockSpec((400, D), lambda i: (i, 0)),
            pl.BlockSpec((D, D), lambda i: (0, 0)),
            pl.BlockSpec((1, D), lambda i: (0, 0)),
        ],
        out_specs=pl.BlockSpec((400, D), lambda i: (i, 0)),
    )(sx, W_read, b_row)


@jax.jit
def kernel(x, segment_ids, W_score, b_score, W_read, b_read):
    del b_score
    ids32 = segment_ids.astype(jnp.int32)
    seg_bounds = jnp.minimum(jnp.arange(33, dtype=jnp.int32) * SPT, NSEG)
    rb = jnp.searchsorted(ids32, seg_bounds, side="left").astype(jnp.int32)
    rb = jnp.concatenate([rb, jnp.full((15,), N, jnp.int32)])
    sx_flat = _sc_segment_softmax_sum(
        x.reshape(N * D), ids32, rb, W_score.reshape(D)
    )
    sx = sx_flat.reshape(NSEG_PAD, D)[:NSEG]
    return _tc_readout(sx, W_read, b_read.reshape(1, D))
